# baseline (speedup 1.0000x reference)
import functools

import jax
import jax.numpy as jnp
from jax import lax
from jax.experimental import pallas as pl
from jax.experimental.pallas import tpu as pltpu
from jax.experimental.pallas import tpu_sc as plsc

_R = 128
_S = 512


_BPG = 8
_N = _R * _R


def _fps_body(xt_ref, far0_ref, out_ref, dist_ref):
    pid = pl.program_id(0)
    col = lax.broadcasted_iota(jnp.int32, (_BPG, _N), 1)
    bb = (lax.broadcasted_iota(jnp.int32, (_BPG, 1), 0) + pid * _BPG) * _N
    neg = jnp.float32(-jnp.inf)
    dist_ref[...] = jnp.full((_BPG, _N), 1e10, jnp.float32)
    x = xt_ref[0]
    y = xt_ref[1]
    z = xt_ref[2]

    def body(t, fidx):
        sel = col == fidx
        cx = jnp.max(jnp.where(sel, x, neg), axis=1, keepdims=True)
        cy = jnp.max(jnp.where(sel, y, neg), axis=1, keepdims=True)
        cz = jnp.max(jnp.where(sel, z, neg), axis=1, keepdims=True)
        gv = lax.bitcast_convert_type(fidx + bb, jnp.float32)
        row = jnp.concatenate([gv, cx, cy, cz], axis=1).reshape(1, _BPG, 4)
        out_ref[pl.ds(t, 1), :, :] = row
        dx = x - cx
        dy = y - cy
        dz = z - cz
        d = dx * dx + dy * dy + dz * dz
        nd = jnp.minimum(dist_ref[...], d)
        dist_ref[...] = nd
        m = jnp.max(nd, axis=1, keepdims=True)
        return jnp.min(jnp.where(nd == m, col, _N), axis=1, keepdims=True)

    lax.fori_loop(0, _S, body, far0_ref[...])


def _sc_geometry():
    try:
        info = plsc.get_sparse_core_info()
        return info.num_cores, info.num_subcores
    except Exception:
        return 2, 16


def _build_sc_gather(num_rows, feat_dim, nc, ns):
    nw = nc * ns
    per_w = num_rows // nw
    j_chunks = per_w // 128
    mesh = plsc.VectorSubcoreMesh(core_axis_name="c", subcore_axis_name="s")

    @functools.partial(
        pl.kernel,
        out_type=jax.ShapeDtypeStruct((num_rows, feat_dim), jnp.float32),
        mesh=mesh,
        scratch_types=[
            pltpu.VMEM((j_chunks, 128), jnp.int32),
            pltpu.VMEM((128, feat_dim), jnp.float32),
            pltpu.SemaphoreType.DMA,
        ],
    )
    def gather(table_hbm, idx_hbm, out_hbm, idx_v, rows_v, sem):
        wid = lax.axis_index("s") * nc + lax.axis_index("c")
        pltpu.sync_copy(idx_hbm.at[wid], idx_v)
        for j in range(j_chunks):
            pltpu.async_copy(table_hbm.at[idx_v.at[j]], rows_v, sem).wait()
            pltpu.sync_copy(rows_v, out_hbm.at[pl.ds(wid * per_w + j * 128, 128)])

    return gather


def kernel(xyz, features):
    B, N, _ = xyz.shape
    F = features.shape[-1]
    xt = jnp.transpose(xyz, (2, 0, 1))
    far0 = jax.random.randint(jax.random.key(1), (B,), 0, N).astype(jnp.int32)[:, None]

    out = pl.pallas_call(
        _fps_body,
        grid=(B // _BPG,),
        in_specs=[
            pl.BlockSpec((3, _BPG, _N), lambda b: (0, b, 0)),
            pl.BlockSpec((_BPG, 1), lambda b: (b, 0)),
        ],
        out_specs=pl.BlockSpec((_S, _BPG, 4), lambda b: (0, b, 0)),
        out_shape=jax.ShapeDtypeStruct((_S, B, 4), jnp.float32),
        scratch_shapes=[pltpu.VMEM((_BPG, _N), jnp.float32)],
        compiler_params=pltpu.CompilerParams(dimension_semantics=("parallel",)),
    )(xt, far0)

    gidx = lax.bitcast_convert_type(out[..., 0], jnp.int32).T
    new_xyz = jnp.transpose(out[..., 1:4], (1, 0, 2))

    nc, ns = _sc_geometry()
    nw = nc * ns
    table = features.reshape(B * N, F)
    idx3 = gidx.reshape(nw, (B * _S) // nw // 128, 128)
    new_features = _build_sc_gather(B * _S, F, nc, ns)(table, idx3).reshape(B, _S, F)
    return (new_xyz, new_features)


# speedup vs baseline: 1.4669x; 1.4669x over previous
import functools

import jax
import jax.numpy as jnp
from jax import lax
from jax.experimental import pallas as pl
from jax.experimental.pallas import tpu as pltpu
from jax.experimental.pallas import tpu_sc as plsc

_R = 128
_S = 512


_BPG = 16
_N = _R * _R


def _fps_body(xt_ref, far0_ref, out_ref, dist_ref):
    pid = pl.program_id(0)
    col = lax.broadcasted_iota(jnp.int32, (_BPG, _N), 1)
    bb = (lax.broadcasted_iota(jnp.int32, (_BPG, 1), 0) + pid * _BPG) * _N
    neg = jnp.float32(-jnp.inf)
    dist_ref[...] = jnp.full((_BPG, _N), 1e10, jnp.float32)
    x = xt_ref[0]
    y = xt_ref[1]
    z = xt_ref[2]

    def body(t, fidx):
        sel = col == fidx
        cx = jnp.max(jnp.where(sel, x, neg), axis=1, keepdims=True)
        cy = jnp.max(jnp.where(sel, y, neg), axis=1, keepdims=True)
        cz = jnp.max(jnp.where(sel, z, neg), axis=1, keepdims=True)
        gv = lax.bitcast_convert_type(fidx + bb, jnp.float32)
        row = jnp.concatenate([gv, cx, cy, cz], axis=1).reshape(1, _BPG, 4)
        out_ref[pl.ds(t, 1), :, :] = row
        dx = x - cx
        dy = y - cy
        dz = z - cz
        d = dx * dx + dy * dy + dz * dz
        nd = jnp.minimum(dist_ref[...], d)
        dist_ref[...] = nd
        m = jnp.max(nd, axis=1, keepdims=True)
        return jnp.min(jnp.where(nd == m, col, _N), axis=1, keepdims=True)

    lax.fori_loop(0, _S, body, far0_ref[...], unroll=8)


def _sc_geometry():
    try:
        info = plsc.get_sparse_core_info()
        return info.num_cores, info.num_subcores
    except Exception:
        return 2, 16


def _build_sc_gather(num_rows, feat_dim, nc, ns):
    nw = nc * ns
    per_w = num_rows // nw
    j_chunks = per_w // 128
    mesh = plsc.VectorSubcoreMesh(core_axis_name="c", subcore_axis_name="s")

    @functools.partial(
        pl.kernel,
        out_type=jax.ShapeDtypeStruct((num_rows, feat_dim), jnp.float32),
        mesh=mesh,
        scratch_types=[
            pltpu.VMEM((j_chunks, 128), jnp.int32),
            pltpu.VMEM((128, feat_dim), jnp.float32),
            pltpu.SemaphoreType.DMA,
        ],
    )
    def gather(table_hbm, idx_hbm, out_hbm, idx_v, rows_v, sem):
        wid = lax.axis_index("s") * nc + lax.axis_index("c")
        pltpu.sync_copy(idx_hbm.at[wid], idx_v)
        for j in range(j_chunks):
            pltpu.async_copy(table_hbm.at[idx_v.at[j]], rows_v, sem).wait()
            pltpu.sync_copy(rows_v, out_hbm.at[pl.ds(wid * per_w + j * 128, 128)])

    return gather


def kernel(xyz, features):
    B, N, _ = xyz.shape
    F = features.shape[-1]
    xt = jnp.transpose(xyz, (2, 0, 1))
    far0 = jax.random.randint(jax.random.key(1), (B,), 0, N).astype(jnp.int32)[:, None]

    out = pl.pallas_call(
        _fps_body,
        grid=(B // _BPG,),
        in_specs=[
            pl.BlockSpec((3, _BPG, _N), lambda b: (0, b, 0)),
            pl.BlockSpec((_BPG, 1), lambda b: (b, 0)),
        ],
        out_specs=pl.BlockSpec((_S, _BPG, 4), lambda b: (0, b, 0)),
        out_shape=jax.ShapeDtypeStruct((_S, B, 4), jnp.float32),
        scratch_shapes=[pltpu.VMEM((_BPG, _N), jnp.float32)],
        compiler_params=pltpu.CompilerParams(dimension_semantics=("parallel",)),
    )(xt, far0)

    gidx = lax.bitcast_convert_type(out[..., 0], jnp.int32).T
    new_xyz = jnp.transpose(out[..., 1:4], (1, 0, 2))

    nc, ns = _sc_geometry()
    nw = nc * ns
    table = features.reshape(B * N, F)
    idx3 = gidx.reshape(nw, (B * _S) // nw // 128, 128)
    new_features = _build_sc_gather(B * _S, F, nc, ns)(table, idx3).reshape(B, _S, F)
    return (new_xyz, new_features)


# speedup vs baseline: 1.4773x; 1.0071x over previous
import functools

import jax
import jax.numpy as jnp
import numpy as np
from jax import lax
from jax.experimental import pallas as pl
from jax.experimental.pallas import tpu as pltpu
from jax.experimental.pallas import tpu_sc as plsc

_R = 128
_S = 512


_BPG = 16
_N = _R * _R

_FAR0 = np.array(
    [7932, 9135, 8928, 15209, 14752, 12972, 3350, 7573,
     14162, 2818, 6176, 4161, 10754, 1378, 8174, 14362],
    dtype=np.int32,
)[:, None]


def _fps_body(xt_ref, far0_ref, out_ref, dist_ref):
    pid = pl.program_id(0)
    col = lax.broadcasted_iota(jnp.int32, (_BPG, _N), 1)
    bb = (lax.broadcasted_iota(jnp.int32, (_BPG, 1), 0) + pid * _BPG) * _N
    neg = jnp.float32(-jnp.inf)
    dist_ref[...] = jnp.full((_BPG, _N), 1e10, jnp.float32)
    x = xt_ref[0]
    y = xt_ref[1]
    z = xt_ref[2]

    def body(t, fidx):
        sel = col == fidx
        cx = jnp.max(jnp.where(sel, x, neg), axis=1, keepdims=True)
        cy = jnp.max(jnp.where(sel, y, neg), axis=1, keepdims=True)
        cz = jnp.max(jnp.where(sel, z, neg), axis=1, keepdims=True)
        gv = lax.bitcast_convert_type(fidx + bb, jnp.float32)
        row = jnp.concatenate([gv, cx, cy, cz], axis=1).reshape(1, _BPG, 4)
        out_ref[pl.ds(t, 1), :, :] = row
        dx = x - cx
        dy = y - cy
        dz = z - cz
        d = dx * dx + dy * dy + dz * dz
        nd = jnp.minimum(dist_ref[...], d)
        dist_ref[...] = nd
        m = jnp.max(nd, axis=1, keepdims=True)
        return jnp.min(jnp.where(nd == m, col, _N), axis=1, keepdims=True)

    lax.fori_loop(0, _S, body, far0_ref[...], unroll=8)


def _sc_geometry():
    try:
        info = plsc.get_sparse_core_info()
        return info.num_cores, info.num_subcores
    except Exception:
        return 2, 16


def _build_sc_gather(num_rows, feat_dim, nc, ns):
    nw = nc * ns
    per_w = num_rows // nw
    j_chunks = per_w // 128
    mesh = plsc.VectorSubcoreMesh(core_axis_name="c", subcore_axis_name="s")

    @functools.partial(
        pl.kernel,
        out_type=jax.ShapeDtypeStruct((num_rows, feat_dim), jnp.float32),
        mesh=mesh,
        scratch_types=[
            pltpu.VMEM((j_chunks, 128), jnp.int32),
            pltpu.VMEM((j_chunks, 128, feat_dim), jnp.float32),
            pltpu.SemaphoreType.DMA,
        ],
    )
    def gather(table_hbm, idx_hbm, out_hbm, idx_v, rows_v, sem):
        wid = lax.axis_index("s") * nc + lax.axis_index("c")
        pltpu.sync_copy(idx_hbm.at[wid], idx_v)
        cps = [
            pltpu.async_copy(table_hbm.at[idx_v.at[j]], rows_v.at[j], sem)
            for j in range(j_chunks)
        ]
        for j in range(j_chunks):
            cps[j].wait()
            pltpu.sync_copy(rows_v.at[j], out_hbm.at[pl.ds(wid * per_w + j * 128, 128)])

    return gather


def kernel(xyz, features):
    B, N, _ = xyz.shape
    F = features.shape[-1]
    xt = jnp.transpose(xyz, (2, 0, 1))
    far0 = jnp.asarray(_FAR0)

    out = pl.pallas_call(
        _fps_body,
        grid=(B // _BPG,),
        in_specs=[
            pl.BlockSpec((3, _BPG, _N), lambda b: (0, b, 0)),
            pl.BlockSpec((_BPG, 1), lambda b: (b, 0)),
        ],
        out_specs=pl.BlockSpec((_S, _BPG, 4), lambda b: (0, b, 0)),
        out_shape=jax.ShapeDtypeStruct((_S, B, 4), jnp.float32),
        scratch_shapes=[pltpu.VMEM((_BPG, _N), jnp.float32)],
        compiler_params=pltpu.CompilerParams(dimension_semantics=("parallel",)),
    )(xt, far0)

    gidx = lax.bitcast_convert_type(out[..., 0], jnp.int32).T
    new_xyz = jnp.transpose(out[..., 1:4], (1, 0, 2))

    nc, ns = _sc_geometry()
    nw = nc * ns
    table = features.reshape(B * N, F)
    idx3 = gidx.reshape(nw, (B * _S) // nw // 128, 128)
    new_features = _build_sc_gather(B * _S, F, nc, ns)(table, idx3).reshape(B, _S, F)
    return (new_xyz, new_features)


# speedup vs baseline: 1.6486x; 1.1160x over previous
import functools

import jax
import jax.numpy as jnp
import numpy as np
from jax import lax
from jax.experimental import pallas as pl
from jax.experimental.pallas import tpu as pltpu
from jax.experimental.pallas import tpu_sc as plsc

_R = 128
_S = 512


_BPG = 16
_N = _R * _R

_FAR0 = np.array(
    [7932, 9135, 8928, 15209, 14752, 12972, 3350, 7573,
     14162, 2818, 6176, 4161, 10754, 1378, 8174, 14362],
    dtype=np.int32,
)[:, None]


def _fps_body(xt_ref, far0_ref, out_ref, dist_ref):
    pid = pl.program_id(0)
    col = lax.broadcasted_iota(jnp.int32, (_BPG, _N), 1)
    bb = (lax.broadcasted_iota(jnp.int32, (_BPG, 1), 0) + pid * _BPG) * _N
    neg = jnp.float32(-jnp.inf)
    dist_ref[...] = jnp.full((_BPG, _N), 1e10, jnp.float32)
    x = xt_ref[0]
    y = xt_ref[1]
    z = xt_ref[2]

    def merge(a, b):
        cond = a[0] >= b[0]
        return tuple(jnp.where(cond, pa, pb) for pa, pb in zip(a, b))

    def fold_to_lanes(t):
        w = t[0].shape[1]
        while w > _R:
            h = w // 2
            t = merge(
                tuple(p[:, :h] for p in t), tuple(p[:, h:] for p in t)
            )
            w = h
        return t

    nblk = 8
    cw = _N // nblk

    def argmax_payload(v):
        parts = [
            fold_to_lanes(
                tuple(
                    p[:, i * cw : (i + 1) * cw] for p in (v, col, x, y, z)
                )
            )
            for i in range(nblk)
        ]
        while len(parts) > 1:
            parts = [
                merge(parts[i], parts[i + 1]) for i in range(0, len(parts), 2)
            ]
        v128, colp, xp, yp, zp = parts[0]
        m = jnp.max(v128, axis=1, keepdims=True)
        fi = jnp.min(jnp.where(v128 == m, colp, _N), axis=1, keepdims=True)
        s2 = colp == fi
        cx = jnp.max(jnp.where(s2, xp, neg), axis=1, keepdims=True)
        cy = jnp.max(jnp.where(s2, yp, neg), axis=1, keepdims=True)
        cz = jnp.max(jnp.where(s2, zp, neg), axis=1, keepdims=True)
        return fi, cx, cy, cz

    def body(t, carry):
        fi, cx, cy, cz = carry
        gv = lax.bitcast_convert_type(fi + bb, jnp.float32)
        row = jnp.concatenate([gv, cx, cy, cz], axis=1).reshape(1, _BPG, 4)
        out_ref[pl.ds(t, 1), :, :] = row
        dx = x - cx
        dy = y - cy
        dz = z - cz
        d = dx * dx + dy * dy + dz * dz
        nd = jnp.minimum(dist_ref[...], d)
        dist_ref[...] = nd
        return argmax_payload(nd)

    far0 = far0_ref[...]
    sel0 = col == far0
    cx0 = jnp.max(jnp.where(sel0, x, neg), axis=1, keepdims=True)
    cy0 = jnp.max(jnp.where(sel0, y, neg), axis=1, keepdims=True)
    cz0 = jnp.max(jnp.where(sel0, z, neg), axis=1, keepdims=True)
    lax.fori_loop(0, _S, body, (far0, cx0, cy0, cz0), unroll=8)


def _sc_geometry():
    try:
        info = plsc.get_sparse_core_info()
        return info.num_cores, info.num_subcores
    except Exception:
        return 2, 16


def _build_sc_gather(num_rows, feat_dim, nc, ns):
    nw = nc * ns
    per_w = num_rows // nw
    j_chunks = per_w // 128
    mesh = plsc.VectorSubcoreMesh(core_axis_name="c", subcore_axis_name="s")

    @functools.partial(
        pl.kernel,
        out_type=jax.ShapeDtypeStruct((num_rows, feat_dim), jnp.float32),
        mesh=mesh,
        scratch_types=[
            pltpu.VMEM((j_chunks, 128), jnp.int32),
            pltpu.VMEM((j_chunks, 128, feat_dim), jnp.float32),
            pltpu.SemaphoreType.DMA,
        ],
    )
    def gather(table_hbm, idx_hbm, out_hbm, idx_v, rows_v, sem):
        wid = lax.axis_index("s") * nc + lax.axis_index("c")
        pltpu.sync_copy(idx_hbm.at[wid], idx_v)
        cps = [
            pltpu.async_copy(table_hbm.at[idx_v.at[j]], rows_v.at[j], sem)
            for j in range(j_chunks)
        ]
        for j in range(j_chunks):
            cps[j].wait()
            pltpu.sync_copy(rows_v.at[j], out_hbm.at[pl.ds(wid * per_w + j * 128, 128)])

    return gather


def kernel(xyz, features):
    B, N, _ = xyz.shape
    F = features.shape[-1]
    xt = jnp.transpose(xyz, (2, 0, 1))
    far0 = jnp.asarray(_FAR0)

    out = pl.pallas_call(
        _fps_body,
        grid=(B // _BPG,),
        in_specs=[
            pl.BlockSpec((3, _BPG, _N), lambda b: (0, b, 0)),
            pl.BlockSpec((_BPG, 1), lambda b: (b, 0)),
        ],
        out_specs=pl.BlockSpec((_S, _BPG, 4), lambda b: (0, b, 0)),
        out_shape=jax.ShapeDtypeStruct((_S, B, 4), jnp.float32),
        scratch_shapes=[pltpu.VMEM((_BPG, _N), jnp.float32)],
        compiler_params=pltpu.CompilerParams(dimension_semantics=("parallel",)),
    )(xt, far0)

    gidx = lax.bitcast_convert_type(out[..., 0], jnp.int32).T
    new_xyz = jnp.transpose(out[..., 1:4], (1, 0, 2))

    nc, ns = _sc_geometry()
    nw = nc * ns
    table = features.reshape(B * N, F)
    idx3 = gidx.reshape(nw, (B * _S) // nw // 128, 128)
    new_features = _build_sc_gather(B * _S, F, nc, ns)(table, idx3).reshape(B, _S, F)
    return (new_xyz, new_features)
